# baseline (speedup 1.0000x reference)
import functools

import jax
import jax.numpy as jnp
from jax import lax
from jax.experimental import pallas as pl
from jax.experimental.pallas import tpu as pltpu
from jax.experimental.pallas import tpu_sc as plsc

_N = 10000
_E = 320000
_D = 128
_H = 128
_HH = _H // 2
_EPS = 1e-3
_INV = 1.0 / (1.0 + _EPS) ** 0.5

_NC = 2
_NS = 16
_EPT = _E // _NS
_C = 125
_NCHUNK = _EPT // _C
_RPT = 632
_NPAD = _RPT * _NS
_ZR = 316
_W = 40
_NWIN = _NCHUNK // _W

_BR = 1000


def _gelu(x):
    return 0.5 * x * (1.0 + lax.erf(x * (2.0 ** -0.5)))


def _ffn1_body(x_ref, g_ref, b_ref, w_ref, bias_ref,
               gx_ref, bx_ref, wx_ref, bias2_ref, lo_ref, hi_ref, zx_ref):
    h = x_ref[...] * (g_ref[...] * _INV) + b_ref[...]
    z = jnp.dot(h, w_ref[...], preferred_element_type=jnp.float32)
    z = _gelu(z + bias_ref[...])
    lo_ref[...] = z[:, :_HH]
    hi_ref[...] = z[:, _HH:]
    hx = x_ref[...] * (gx_ref[...] * _INV) + bx_ref[...]
    zx_ref[...] = jnp.dot(hx, wx_ref[...],
                          preferred_element_type=jnp.float32) + bias2_ref[...]


def _ffn1(x, gamma1, beta1, w1, b1, gamma2, beta2, w2, b2):
    full = lambda shape: pl.BlockSpec(shape, lambda i: (0, 0))
    return pl.pallas_call(
        _ffn1_body,
        grid=(_N // _BR,),
        in_specs=[
            pl.BlockSpec((_BR, _D), lambda i: (i, 0)),
            full((1, _D)), full((1, _D)), full((_D, _H)), full((1, _H)),
            full((1, _D)), full((1, _D)), full((_D, _H)), full((1, _H)),
        ],
        out_specs=[pl.BlockSpec((_BR, _HH), lambda i: (i, 0)),
                   pl.BlockSpec((_BR, _HH), lambda i: (i, 0)),
                   pl.BlockSpec((_BR, _H), lambda i: (i, 0))],
        out_shape=[jax.ShapeDtypeStruct((_N, _HH), jnp.float32),
                   jax.ShapeDtypeStruct((_N, _HH), jnp.float32),
                   jax.ShapeDtypeStruct((_N, _H), jnp.float32)],
    )(x, gamma1.reshape(1, _D), beta1.reshape(1, _D), w1, b1.reshape(1, _H),
      gamma2[:_D].reshape(1, _D), beta2[:_D].reshape(1, _D), w2[:_D],
      b2.reshape(1, _H))


def _sc_aggregate(prep_lo, prep_hi, dst3, src3, w2):
    mesh = plsc.VectorSubcoreMesh(core_axis_name="c", subcore_axis_name="s")

    @functools.partial(
        pl.kernel,
        mesh=mesh,
        compiler_params=pltpu.CompilerParams(use_tc_tiling_on_sc=False),
        out_type=[
            jax.ShapeDtypeStruct((_NC, _NPAD, _HH), jnp.float32),
            jax.ShapeDtypeStruct((_NC, _NPAD, 16), jnp.float32),
        ],
        scratch_types=[
            pltpu.VMEM((_W, 1, _C), jnp.int32),
            pltpu.VMEM((_W, 1, _C), jnp.int32),
            pltpu.VMEM((_W, 128), jnp.float32),
            pltpu.VMEM((_C, _HH), jnp.float32),
            pltpu.VMEM((_C, _HH), jnp.float32),
            pltpu.VMEM((_C, _HH), jnp.float32),
            pltpu.VMEM((_C, _HH), jnp.float32),
            pltpu.VMEM((_C, 16), jnp.float32),
            pltpu.VMEM((_ZR, _HH), jnp.float32),
            pltpu.VMEM((_ZR, 16), jnp.float32),
            pltpu.VMEM_SHARED((_NPAD, _HH), jnp.float32),
            pltpu.VMEM_SHARED((_NPAD, 16), jnp.float32),
            pltpu.SemaphoreType.DMA,
            pltpu.SemaphoreType.DMA,
            pltpu.SemaphoreType.DMA,
            pltpu.SemaphoreType.DMA,
            pltpu.SemaphoreType.DMA,
        ],
    )
    def body(lo_hbm, hi_hbm, dst_hbm, src_hbm, w_hbm, acc_out, cnt_out,
             dv, sv, wv, gb0, gb1, sb0, sb1, ones_v, zrow, zcnt,
             acc_sh, cnt_sh, gsem0, gsem1, ssem0, ssem1, csem):
        cid = lax.axis_index("c")
        sid = lax.axis_index("s")
        gb = (gb0, gb1)
        sb = (sb0, sb1)
        gsem = (gsem0, gsem1)
        ssem = (ssem0, ssem1)

        def const_fill(i, _):
            ones_v[i, :] = jnp.ones((16,), jnp.float32)
            return 0

        lax.fori_loop(0, _C, const_fill, 0)

        def zero_fill(i, _):
            for r in range(_HH // 16):
                zrow[i, pl.ds(r * 16, 16)] = jnp.zeros((16,), jnp.float32)
            zcnt[i, :] = jnp.zeros((16,), jnp.float32)
            return 0

        lax.fori_loop(0, _ZR, zero_fill, 0)

        base = sid * _RPT
        for z in range(_RPT // _ZR):
            o = base + z * _ZR
            pltpu.sync_copy(zrow, acc_sh.at[pl.ds(o, _ZR)])
            pltpu.sync_copy(zcnt, cnt_sh.at[pl.ds(o, _ZR)])
        plsc.subcore_barrier()

        def issue_gather(k, buf, sem):
            @pl.when(cid == 0)
            def _():
                pltpu.async_copy(lo_hbm.at[dv.at[k, 0]], buf, sem)

            @pl.when(cid == 1)
            def _():
                pltpu.async_copy(hi_hbm.at[dv.at[k, 0]], buf, sem)

        def window(wi, _):
            cb = sid * _NCHUNK + wi * _W
            pltpu.sync_copy(dst_hbm.at[pl.ds(cb, _W)], dv)
            pltpu.sync_copy(src_hbm.at[pl.ds(cb, _W)], sv)
            pltpu.sync_copy(w_hbm.at[pl.ds(cb, _W)], wv)

            issue_gather(0, gb0, gsem0)
            issue_gather(1, gb1, gsem1)

            def outer(kk, _):
                for b in range(2):
                    k = kk * 2 + b
                    gbuf = gb[b]
                    sbuf = sb[b]
                    pltpu.make_async_copy(
                        lo_hbm.at[dv.at[k, 0]], gbuf, gsem[b]).wait()

                    @pl.when(k >= 2)
                    def _():
                        pltpu.make_async_copy(
                            sbuf, acc_sh.at[sv.at[k, 0]], ssem[b]).wait()

                    for g in range(-(-_C // 16)):
                        wg = wv[k, pl.ds(g * 16, 16)]
                        for j in range(min(16, _C - g * 16)):
                            ws = jnp.full((16,), wg[j], jnp.float32)
                            i = g * 16 + j
                            for r in range(_HH // 16):
                                cs = pl.ds(r * 16, 16)
                                sbuf[i, cs] = gbuf[i, cs] * ws

                    @pl.when(k + 2 < _W)
                    def _():
                        issue_gather(k + 2, gbuf, gsem[b])

                    pltpu.async_copy(sbuf, acc_sh.at[sv.at[k, 0]], ssem[b],
                                     add=True)

                    @pl.when((cid == b) & (k >= 2))
                    def _():
                        pltpu.make_async_copy(
                            ones_v, cnt_sh.at[sv.at[k, 0]], csem).wait()

                    @pl.when(cid == b)
                    def _():
                        pltpu.async_copy(ones_v, cnt_sh.at[sv.at[k, 0]],
                                         csem, add=True)
                return 0

            lax.fori_loop(0, _W // 2, outer, 0)

            for b in range(2):
                pltpu.make_async_copy(
                    sb[b], acc_sh.at[sv.at[0, 0]], ssem[b]).wait()
            pltpu.make_async_copy(ones_v, cnt_sh.at[sv.at[0, 0]], csem).wait()
            return 0

        lax.fori_loop(0, _NWIN, window, 0)
        plsc.subcore_barrier()

        for z in range(_RPT // _ZR):
            o = base + z * _ZR
            pltpu.sync_copy(acc_sh.at[pl.ds(o, _ZR)],
                            acc_out.at[cid, pl.ds(o, _ZR)])
            pltpu.sync_copy(cnt_sh.at[pl.ds(o, _ZR)],
                            cnt_out.at[cid, pl.ds(o, _ZR)])

    return body(prep_lo, prep_hi, dst3, src3, w2)


def _ffn2_body(zx_ref, a0_ref, a1_ref, c0_ref, c1_ref,
               ga_ref, ba_ref, wlo_ref, whi_ref, o_ref):
    cnt = jnp.maximum(c0_ref[...][:, 0:1] + c1_ref[...][:, 0:1], 1.0)
    agg_lo = a0_ref[...] / cnt
    agg_hi = a1_ref[...] / cnt
    ga = ga_ref[...] * _INV
    ba = ba_ref[...]
    hlo = agg_lo * ga[:, :_HH] + ba[:, :_HH]
    hhi = agg_hi * ga[:, _HH:] + ba[:, _HH:]
    z = (zx_ref[...]
         + jnp.dot(hlo, wlo_ref[...], preferred_element_type=jnp.float32)
         + jnp.dot(hhi, whi_ref[...], preferred_element_type=jnp.float32))
    o_ref[...] = _gelu(z)


def _ffn2(zx, acc2, cnt2, gamma2, beta2, w2):
    full = lambda shape: pl.BlockSpec(shape, lambda i: (0, 0))
    row = lambda width: pl.BlockSpec((_BR, width), lambda i: (i, 0))
    return pl.pallas_call(
        _ffn2_body,
        grid=(_N // _BR,),
        in_specs=[
            row(_H), row(_HH), row(_HH), row(16), row(16),
            full((1, _H)), full((1, _H)),
            full((_HH, _H)), full((_HH, _H)),
        ],
        out_specs=pl.BlockSpec((_BR, _H), lambda i: (i, 0)),
        out_shape=jax.ShapeDtypeStruct((_N, _H), jnp.float32),
    )(zx, acc2[0], acc2[1], cnt2[0], cnt2[1],
      gamma2[_D:].reshape(1, _H), beta2[_D:].reshape(1, _H),
      w2[_D:_D + _HH], w2[_D + _HH:])


def kernel(node_representations, edges, edge_weights,
           gamma1, beta1, W1, b1, gamma2, beta2, W2, b2):
    x = node_representations
    nch = _E // _C
    dst3 = edges[1].astype(jnp.int32).reshape(nch, 1, _C)
    src3 = edges[0].astype(jnp.int32).reshape(nch, 1, _C)
    w2 = jnp.pad(edge_weights.astype(jnp.float32).reshape(nch, _C),
                 ((0, 0), (0, 128 - _C)))

    prep_lo, prep_hi, zx = _ffn1(x, gamma1, beta1, W1, b1,
                                 gamma2, beta2, W2, b2)
    acc2, cnt2 = _sc_aggregate(prep_lo, prep_hi, dst3, src3, w2)
    return _ffn2(zx, acc2, cnt2, gamma2, beta2, W2)


# speedup vs baseline: 1.0240x; 1.0240x over previous
import functools

import jax
import jax.numpy as jnp
from jax import lax
from jax.experimental import pallas as pl
from jax.experimental.pallas import tpu as pltpu
from jax.experimental.pallas import tpu_sc as plsc

_N = 10000
_E = 320000
_D = 128
_H = 128
_HH = _H // 2
_EPS = 1e-3
_INV = 1.0 / (1.0 + _EPS) ** 0.5

_NC = 2
_NS = 16
_EPT = _E // _NS
_C = 125
_NCHUNK = _EPT // _C
_RPT = 632
_NPAD = _RPT * _NS
_ZR = 316
_W = 40
_NWIN = _NCHUNK // _W

_BR = 1000


def _gelu(x):
    return 0.5 * x * (1.0 + lax.erf(x * (2.0 ** -0.5)))


def _ffn1_body(x_ref, g_ref, b_ref, w_ref, bias_ref, lo_ref, hi_ref):
    h = x_ref[...] * (g_ref[...] * _INV) + b_ref[...]
    z = jnp.dot(h, w_ref[...], preferred_element_type=jnp.float32)
    z = _gelu(z + bias_ref[...])
    lo_ref[...] = z[:, :_HH]
    hi_ref[...] = z[:, _HH:]


def _ffn1(x, gamma1, beta1, w1, b1):
    full = lambda shape: pl.BlockSpec(shape, lambda i: (0, 0))
    return pl.pallas_call(
        _ffn1_body,
        grid=(_N // _BR,),
        in_specs=[
            pl.BlockSpec((_BR, _D), lambda i: (i, 0)),
            full((1, _D)), full((1, _D)), full((_D, _H)), full((1, _H)),
        ],
        out_specs=[pl.BlockSpec((_BR, _HH), lambda i: (i, 0)),
                   pl.BlockSpec((_BR, _HH), lambda i: (i, 0))],
        out_shape=[jax.ShapeDtypeStruct((_N, _HH), jnp.float32),
                   jax.ShapeDtypeStruct((_N, _HH), jnp.float32)],
    )(x, gamma1.reshape(1, _D), beta1.reshape(1, _D), w1, b1.reshape(1, _H))


def _sc_aggregate(prep_lo, prep_hi, dst3, src3, w2):
    mesh = plsc.VectorSubcoreMesh(core_axis_name="c", subcore_axis_name="s")

    @functools.partial(
        pl.kernel,
        mesh=mesh,
        compiler_params=pltpu.CompilerParams(use_tc_tiling_on_sc=False),
        out_type=[
            jax.ShapeDtypeStruct((_NC, _NPAD, _HH), jnp.float32),
            jax.ShapeDtypeStruct((_NC, _NPAD, 16), jnp.float32),
        ],
        scratch_types=[
            pltpu.VMEM((_W, 1, _C), jnp.int32),
            pltpu.VMEM((_W, 1, _C), jnp.int32),
            pltpu.VMEM((_W, 128), jnp.float32),
            pltpu.VMEM((_C, _HH), jnp.float32),
            pltpu.VMEM((_C, _HH), jnp.float32),
            pltpu.VMEM((_C, _HH), jnp.float32),
            pltpu.VMEM((_C, _HH), jnp.float32),
            pltpu.VMEM((_C, 16), jnp.float32),
            pltpu.VMEM((_ZR, _HH), jnp.float32),
            pltpu.VMEM((_ZR, 16), jnp.float32),
            pltpu.VMEM_SHARED((_NPAD, _HH), jnp.float32),
            pltpu.VMEM_SHARED((_NPAD, 16), jnp.float32),
            pltpu.SemaphoreType.DMA,
            pltpu.SemaphoreType.DMA,
            pltpu.SemaphoreType.DMA,
            pltpu.SemaphoreType.DMA,
            pltpu.SemaphoreType.DMA,
        ],
    )
    def body(lo_hbm, hi_hbm, dst_hbm, src_hbm, w_hbm, acc_out, cnt_out,
             dv, sv, wv, gb0, gb1, sb0, sb1, ones_v, zrow, zcnt,
             acc_sh, cnt_sh, gsem0, gsem1, ssem0, ssem1, csem):
        cid = lax.axis_index("c")
        sid = lax.axis_index("s")
        gb = (gb0, gb1)
        sb = (sb0, sb1)
        gsem = (gsem0, gsem1)
        ssem = (ssem0, ssem1)

        def const_fill(i, _):
            ones_v[i, :] = jnp.ones((16,), jnp.float32)
            return 0

        lax.fori_loop(0, _C, const_fill, 0)

        def zero_fill(i, _):
            for r in range(_HH // 16):
                zrow[i, pl.ds(r * 16, 16)] = jnp.zeros((16,), jnp.float32)
            zcnt[i, :] = jnp.zeros((16,), jnp.float32)
            return 0

        lax.fori_loop(0, _ZR, zero_fill, 0)

        base = sid * _RPT
        for z in range(_RPT // _ZR):
            o = base + z * _ZR
            pltpu.sync_copy(zrow, acc_sh.at[pl.ds(o, _ZR)])
            pltpu.sync_copy(zcnt, cnt_sh.at[pl.ds(o, _ZR)])
        plsc.subcore_barrier()

        def issue_gather(k, buf, sem):
            @pl.when(cid == 0)
            def _():
                pltpu.async_copy(lo_hbm.at[dv.at[k, 0]], buf, sem)

            @pl.when(cid == 1)
            def _():
                pltpu.async_copy(hi_hbm.at[dv.at[k, 0]], buf, sem)

        def window(wi, _):
            cb = sid * _NCHUNK + wi * _W
            pltpu.sync_copy(dst_hbm.at[pl.ds(cb, _W)], dv)
            pltpu.sync_copy(src_hbm.at[pl.ds(cb, _W)], sv)
            pltpu.sync_copy(w_hbm.at[pl.ds(cb, _W)], wv)

            issue_gather(0, gb0, gsem0)
            issue_gather(1, gb1, gsem1)

            def outer(kk, _):
                for b in range(2):
                    k = kk * 2 + b
                    gbuf = gb[b]
                    sbuf = sb[b]
                    pltpu.make_async_copy(
                        lo_hbm.at[dv.at[k, 0]], gbuf, gsem[b]).wait()

                    @pl.when(k >= 2)
                    def _():
                        pltpu.make_async_copy(
                            sbuf, acc_sh.at[sv.at[k, 0]], ssem[b]).wait()

                    for g in range(-(-_C // 16)):
                        wg = wv[k, pl.ds(g * 16, 16)]
                        for j in range(min(16, _C - g * 16)):
                            ws = jnp.full((16,), wg[j], jnp.float32)
                            i = g * 16 + j
                            for r in range(_HH // 16):
                                cs = pl.ds(r * 16, 16)
                                sbuf[i, cs] = gbuf[i, cs] * ws

                    @pl.when(k + 2 < _W)
                    def _():
                        issue_gather(k + 2, gbuf, gsem[b])

                    pltpu.async_copy(sbuf, acc_sh.at[sv.at[k, 0]], ssem[b],
                                     add=True)

                    @pl.when((cid == b) & (k >= 2))
                    def _():
                        pltpu.make_async_copy(
                            ones_v, cnt_sh.at[sv.at[k, 0]], csem).wait()

                    @pl.when(cid == b)
                    def _():
                        pltpu.async_copy(ones_v, cnt_sh.at[sv.at[k, 0]],
                                         csem, add=True)
                return 0

            lax.fori_loop(0, _W // 2, outer, 0)

            for b in range(2):
                pltpu.make_async_copy(
                    sb[b], acc_sh.at[sv.at[0, 0]], ssem[b]).wait()
            pltpu.make_async_copy(ones_v, cnt_sh.at[sv.at[0, 0]], csem).wait()
            return 0

        lax.fori_loop(0, _NWIN, window, 0)
        plsc.subcore_barrier()

        for z in range(_RPT // _ZR):
            o = base + z * _ZR
            pltpu.sync_copy(acc_sh.at[pl.ds(o, _ZR)],
                            acc_out.at[cid, pl.ds(o, _ZR)])
            pltpu.sync_copy(cnt_sh.at[pl.ds(o, _ZR)],
                            cnt_out.at[cid, pl.ds(o, _ZR)])

    return body(prep_lo, prep_hi, dst3, src3, w2)


def _ffn2_body(x_ref, a0_ref, a1_ref, c0_ref, c1_ref, gx_ref, bx_ref,
               ga_ref, ba_ref, wx_ref, wlo_ref, whi_ref, bias_ref, o_ref):
    cnt = jnp.maximum(c0_ref[...][:, 0:1] + c1_ref[...][:, 0:1], 1.0)
    agg_lo = a0_ref[...] / cnt
    agg_hi = a1_ref[...] / cnt
    hx = x_ref[...] * (gx_ref[...] * _INV) + bx_ref[...]
    ga = ga_ref[...] * _INV
    ba = ba_ref[...]
    hlo = agg_lo * ga[:, :_HH] + ba[:, :_HH]
    hhi = agg_hi * ga[:, _HH:] + ba[:, _HH:]
    z = (jnp.dot(hx, wx_ref[...], preferred_element_type=jnp.float32)
         + jnp.dot(hlo, wlo_ref[...], preferred_element_type=jnp.float32)
         + jnp.dot(hhi, whi_ref[...], preferred_element_type=jnp.float32))
    o_ref[...] = _gelu(z + bias_ref[...])


def _ffn2(x, acc2, cnt2, gamma2, beta2, w2, b2):
    full = lambda shape: pl.BlockSpec(shape, lambda i: (0, 0))
    row = lambda width: pl.BlockSpec((_BR, width), lambda i: (i, 0))
    return pl.pallas_call(
        _ffn2_body,
        grid=(_N // _BR,),
        in_specs=[
            row(_D), row(_HH), row(_HH), row(16), row(16),
            full((1, _D)), full((1, _D)), full((1, _H)), full((1, _H)),
            full((_D, _H)), full((_HH, _H)), full((_HH, _H)), full((1, _H)),
        ],
        out_specs=pl.BlockSpec((_BR, _H), lambda i: (i, 0)),
        out_shape=jax.ShapeDtypeStruct((_N, _H), jnp.float32),
    )(x, acc2[0], acc2[1], cnt2[0], cnt2[1],
      gamma2[:_D].reshape(1, _D), beta2[:_D].reshape(1, _D),
      gamma2[_D:].reshape(1, _H), beta2[_D:].reshape(1, _H),
      w2[:_D], w2[_D:_D + _HH], w2[_D + _HH:], b2.reshape(1, _H))


def kernel(node_representations, edges, edge_weights,
           gamma1, beta1, W1, b1, gamma2, beta2, W2, b2):
    x = node_representations
    nch = _E // _C
    dst3 = edges[1].astype(jnp.int32).reshape(nch, 1, _C)
    src3 = edges[0].astype(jnp.int32).reshape(nch, 1, _C)
    w2 = jnp.pad(edge_weights.astype(jnp.float32).reshape(nch, _C),
                 ((0, 0), (0, 128 - _C)))

    prep_lo, prep_hi = _ffn1(x, gamma1, beta1, W1, b1)
    acc2, cnt2 = _sc_aggregate(prep_lo, prep_hi, dst3, src3, w2)
    return _ffn2(x, acc2, cnt2, gamma2, beta2, W2, b2)
